# baseline (speedup 1.0000x reference)
import functools

import jax
import jax.numpy as jnp
from jax import lax
from jax.experimental import pallas as pl
from jax.experimental.pallas import tpu as pltpu
from jax.experimental.pallas import tpu_sc as plsc

N = 10000
E = 160000
D = 256
DH = 128

NC = 2
NS = 16
LANES = 16

NPAD = 10240
ROWS_PER_TILE = NPAD // NS
ROW_CHUNKS = ROWS_PER_TILE // 128
EC = 128
CHUNKS_PER_TILE = 80
EPT = CHUNKS_PER_TILE * EC
EPAD = NS * EPT
PAD_NODE = N

_MESH = plsc.VectorSubcoreMesh(core_axis_name="c", subcore_axis_name="s")


def _fill2d(ref, nrows, ncolchunks, val):
    v = jnp.full((LANES,), val, dtype=jnp.float32)

    def body(i, carry):
        for cc in range(ncolchunks):
            ref[i, pl.ds(cc * LANES, LANES)] = v
        return carry

    lax.fori_loop(0, nrows, body, 0)


DEG_CPT = NS * CHUNKS_PER_TILE // (NC * NS)


def _fill_lane(ref, lane):
    i16 = lax.iota(jnp.int32, LANES)

    def body(i, carry):
        for cc in range(DH // LANES):
            v = jnp.where(i16 + cc * LANES == lane, jnp.float32(1.0),
                          jnp.float32(0.0))
            ref[i, pl.ds(cc * LANES, LANES)] = v
        return carry

    lax.fori_loop(0, EC, body, 0)


@functools.partial(
    pl.kernel,
    out_type=jax.ShapeDtypeStruct((NC, NPAD, DH), jnp.float32),
    mesh=_MESH,
    scratch_types=[
        pltpu.VMEM((DEG_CPT, EC), jnp.int32),
        pltpu.VMEM((DEG_CPT, EC), jnp.int32),
        pltpu.VMEM((EC, DH), jnp.float32),
        pltpu.VMEM((EC, DH), jnp.float32),
        pltpu.VMEM_SHARED((NPAD, DH), jnp.float32),
    ],
)
def _deg_kernel(edges3, deg3, idxs_v, idxd_v, bufa_v, bufb_v, shared):
    c = lax.axis_index("c")
    s = lax.axis_index("s")

    _fill2d(bufa_v, EC, DH // LANES, 0.0)
    for t in range(ROW_CHUNKS):
        pltpu.sync_copy(bufa_v, shared.at[pl.ds(s * ROWS_PER_TILE + t * 128, 128)])

    base = (c * NS + s) * DEG_CPT
    pltpu.sync_copy(edges3.at[0, pl.ds(base, DEG_CPT)], idxs_v)
    pltpu.sync_copy(edges3.at[1, pl.ds(base, DEG_CPT)], idxd_v)
    _fill_lane(bufa_v, 0)
    _fill_lane(bufb_v, 1)
    plsc.subcore_barrier()

    def body(j, carry):
        pltpu.sync_copy(bufa_v, shared.at[idxs_v.at[j]], add=True)
        pltpu.sync_copy(bufb_v, shared.at[idxd_v.at[j]], add=True)
        return carry

    lax.fori_loop(0, DEG_CPT, body, 0)
    plsc.subcore_barrier()

    for t in range(ROW_CHUNKS):
        r0 = s * ROWS_PER_TILE + t * 128
        pltpu.sync_copy(shared.at[pl.ds(r0, 128)], bufa_v)
        pltpu.sync_copy(bufa_v, deg3.at[c, pl.ds(r0, 128)])


NBUF = 2
HALF = CHUNKS_PER_TILE // 2
HGROUPS = HALF // NBUF


@functools.partial(
    pl.kernel,
    out_type=jax.ShapeDtypeStruct((NC, NPAD, DH), jnp.float32),
    mesh=_MESH,
    scratch_types=[
        pltpu.VMEM((HALF, EC), jnp.int32),
        pltpu.VMEM((HALF, EC), jnp.int32),
        pltpu.VMEM((NBUF * EC, DH), jnp.float32),
        pltpu.VMEM_SHARED((NPAD, DH), jnp.float32),
        pltpu.SemaphoreType.DMA,
        pltpu.SemaphoreType.DMA,
        pltpu.SemaphoreType.DMA,
        pltpu.SemaphoreType.DMA,
    ],
)
def _agg_kernel(hs3, edges3, agg3, idxs_v, idxd_v, rows_v, shared,
                sem0, sem1, ssem0, ssem1):
    c = lax.axis_index("c")
    s = lax.axis_index("s")
    sems = (sem0, sem1)
    ssems = (ssem0, ssem1)

    def buf(b):
        return rows_v.at[pl.ds(b * EC, EC)]

    _fill2d(rows_v, EC, DH // LANES, 0.0)
    for t in range(ROW_CHUNKS):
        pltpu.sync_copy(buf(0), shared.at[pl.ds(s * ROWS_PER_TILE + t * 128, 128)])
    plsc.subcore_barrier()

    for h in range(2):
        base = s * CHUNKS_PER_TILE + h * HALF
        pltpu.sync_copy(edges3.at[0, pl.ds(base, HALF)], idxs_v)
        pltpu.sync_copy(edges3.at[1, pl.ds(base, HALF)], idxd_v)

        for b in range(NBUF):
            pltpu.async_copy(hs3.at[c].at[idxs_v.at[b]], buf(b), sems[b])

        def body(g, carry):
            j0 = g * NBUF
            for b in range(NBUF):
                j = j0 + b
                pltpu.make_async_copy(hs3.at[c].at[idxs_v.at[j]], buf(b), sems[b]).wait()
                pltpu.async_copy(buf(b), shared.at[idxd_v.at[j]], ssems[b], add=True)
            for b in range(NBUF):
                j = j0 + b
                pltpu.make_async_copy(buf(b), shared.at[idxd_v.at[j]], ssems[b]).wait()
                pltpu.async_copy(hs3.at[c].at[idxs_v.at[j + NBUF]], buf(b), sems[b])
            return carry

        lax.fori_loop(0, HGROUPS - 1, body, 0)
        j0 = (HGROUPS - 1) * NBUF
        for b in range(NBUF):
            j = j0 + b
            pltpu.make_async_copy(hs3.at[c].at[idxs_v.at[j]], buf(b), sems[b]).wait()
            pltpu.async_copy(buf(b), shared.at[idxd_v.at[j]], ssems[b], add=True)
        for b in range(NBUF):
            j = j0 + b
            pltpu.make_async_copy(buf(b), shared.at[idxd_v.at[j]], ssems[b]).wait()

    plsc.subcore_barrier()

    for t in range(ROW_CHUNKS):
        r0 = s * ROWS_PER_TILE + t * 128
        pltpu.sync_copy(shared.at[pl.ds(r0, 128)], buf(0))
        pltpu.sync_copy(buf(0), agg3.at[c, pl.ds(r0, 128)])


def _pre_body(deg_ref, x_ref, ns_ref, nd_ref, hs_ref):
    do = deg_ref[0, :, 0:1] + deg_ref[1, :, 0:1]
    di = deg_ref[0, :, 1:2] + deg_ref[1, :, 1:2]
    ns = jnp.where(do > 0, lax.rsqrt(do), 0.0)
    nd = jnp.where(di > 0, lax.rsqrt(di), 0.0)
    ns_ref[...] = jnp.broadcast_to(ns, (128, DH))
    nd_ref[...] = jnp.broadcast_to(nd, (128, DH))
    hs = x_ref[...] * ns
    hs_ref[0] = hs[:, :DH]
    hs_ref[1] = hs[:, DH:]


_pre_call = pl.pallas_call(
    _pre_body,
    grid=(NPAD // 128,),
    in_specs=[
        pl.BlockSpec((NC, 128, DH), lambda i: (0, i, 0)),
        pl.BlockSpec((128, D), lambda i: (i, 0)),
    ],
    out_specs=[
        pl.BlockSpec((128, DH), lambda i: (i, 0)),
        pl.BlockSpec((128, DH), lambda i: (i, 0)),
        pl.BlockSpec((NC, 128, DH), lambda i: (0, i, 0)),
    ],
    out_shape=[
        jax.ShapeDtypeStruct((NPAD, DH), jnp.float32),
        jax.ShapeDtypeStruct((NPAD, DH), jnp.float32),
        jax.ShapeDtypeStruct((NC, NPAD, DH), jnp.float32),
    ],
)


def _layer_body(agg_ref, nd_ref, ns_ref, w_ref, b_ref, hs_ref):
    w = w_ref[...]
    t = jnp.dot(agg_ref[0], w[:DH, :], preferred_element_type=jnp.float32)
    t += jnp.dot(agg_ref[1], w[DH:, :], preferred_element_type=jnp.float32)
    t = t * nd_ref[:, :1]
    h = jnp.maximum(t + b_ref[...], 0.0)
    hs = h * ns_ref[:, :1]
    hs_ref[0] = hs[:, :DH]
    hs_ref[1] = hs[:, DH:]


_layer_call = pl.pallas_call(
    _layer_body,
    grid=(NPAD // 128,),
    in_specs=[
        pl.BlockSpec((NC, 128, DH), lambda i: (0, i, 0)),
        pl.BlockSpec((128, DH), lambda i: (i, 0)),
        pl.BlockSpec((128, DH), lambda i: (i, 0)),
        pl.BlockSpec((D, D), lambda i: (0, 0)),
        pl.BlockSpec((1, D), lambda i: (0, 0)),
    ],
    out_specs=[
        pl.BlockSpec((NC, 128, DH), lambda i: (0, i, 0)),
    ],
    out_shape=[
        jax.ShapeDtypeStruct((NC, NPAD, DH), jnp.float32),
    ],
)


def _final_body(agg_ref, nd_ref, w_ref, b_ref, h_ref, hc_ref):
    w = w_ref[...]
    t = jnp.dot(agg_ref[0], w[:DH, :], preferred_element_type=jnp.float32)
    t += jnp.dot(agg_ref[1], w[DH:, :], preferred_element_type=jnp.float32)
    t = t * nd_ref[:, :1]
    h = jnp.maximum(t + b_ref[...], 0.0)
    h_ref[...] = h
    hc_ref[...] = jnp.where(h >= 0.5, jnp.float32(1.0), jnp.float32(0.0))


_final_call = pl.pallas_call(
    _final_body,
    grid=(NPAD // 128,),
    in_specs=[
        pl.BlockSpec((NC, 128, DH), lambda i: (0, i, 0)),
        pl.BlockSpec((128, DH), lambda i: (i, 0)),
        pl.BlockSpec((D, D), lambda i: (0, 0)),
        pl.BlockSpec((1, D), lambda i: (0, 0)),
    ],
    out_specs=[
        pl.BlockSpec((128, D), lambda i: (i, 0)),
        pl.BlockSpec((128, D), lambda i: (i, 0)),
    ],
    out_shape=[
        jax.ShapeDtypeStruct((NPAD, D), jnp.float32),
        jax.ShapeDtypeStruct((NPAD, D), jnp.float32),
    ],
)


def kernel(x, edge_index, W1, W2, W3, W4, W5, b1, b2, b3, b4, b5):
    epad = jnp.full((2, EPAD - E), PAD_NODE, dtype=jnp.int32)
    edges3 = jnp.concatenate([edge_index, epad], axis=1).reshape(
        2, NS * CHUNKS_PER_TILE, EC)
    xp = jnp.pad(x, ((0, NPAD - N), (0, 0)))

    deg3 = _deg_kernel(edges3)
    ns, nd, hs3 = _pre_call(deg3, xp)

    for W, b in ((W1, b1), (W2, b2), (W3, b3), (W4, b4)):
        agg3 = _agg_kernel(hs3, edges3)
        (hs3,) = _layer_call(agg3, nd, ns, W, b.reshape(1, D))

    agg3 = _agg_kernel(hs3, edges3)
    h, hc = _final_call(agg3, nd, W5, b5.reshape(1, D))
    return h[:N], hc[:N]


# speedup vs baseline: 1.0807x; 1.0807x over previous
import functools

import jax
import jax.numpy as jnp
from jax import lax
from jax.experimental import pallas as pl
from jax.experimental.pallas import tpu as pltpu
from jax.experimental.pallas import tpu_sc as plsc

N = 10000
E = 160000
D = 256
DH = 128

NC = 2
NS = 16
LANES = 16

NPAD = 10240
ROWS_PER_TILE = NPAD // NS
ROW_CHUNKS = ROWS_PER_TILE // 128
EC = 128
CHUNKS_PER_TILE = 80
EPT = CHUNKS_PER_TILE * EC
EPAD = NS * EPT
PAD_NODE = N

_MESH = plsc.VectorSubcoreMesh(core_axis_name="c", subcore_axis_name="s")


def _fill2d(ref, nrows, ncolchunks, val):
    v = jnp.full((LANES,), val, dtype=jnp.float32)

    def body(i, carry):
        for cc in range(ncolchunks):
            ref[i, pl.ds(cc * LANES, LANES)] = v
        return carry

    lax.fori_loop(0, nrows, body, 0)


DEG_CPT = NS * CHUNKS_PER_TILE // (NC * NS)


def _fill_lane(ref, lane):
    i16 = lax.iota(jnp.int32, LANES)

    def body(i, carry):
        for cc in range(DH // LANES):
            v = jnp.where(i16 + cc * LANES == lane, jnp.float32(1.0),
                          jnp.float32(0.0))
            ref[i, pl.ds(cc * LANES, LANES)] = v
        return carry

    lax.fori_loop(0, EC, body, 0)


@functools.partial(
    pl.kernel,
    out_type=jax.ShapeDtypeStruct((NC, NPAD, DH), jnp.float32),
    mesh=_MESH,
    scratch_types=[
        pltpu.VMEM((DEG_CPT, EC), jnp.int32),
        pltpu.VMEM((DEG_CPT, EC), jnp.int32),
        pltpu.VMEM((EC, DH), jnp.float32),
        pltpu.VMEM((EC, DH), jnp.float32),
        pltpu.VMEM_SHARED((NPAD, DH), jnp.float32),
    ],
)
def _deg_kernel(edges3, deg3, idxs_v, idxd_v, bufa_v, bufb_v, shared):
    c = lax.axis_index("c")
    s = lax.axis_index("s")

    _fill2d(bufa_v, EC, DH // LANES, 0.0)
    for t in range(ROW_CHUNKS):
        pltpu.sync_copy(bufa_v, shared.at[pl.ds(s * ROWS_PER_TILE + t * 128, 128)])

    base = (c * NS + s) * DEG_CPT
    pltpu.sync_copy(edges3.at[0, pl.ds(base, DEG_CPT)], idxs_v)
    pltpu.sync_copy(edges3.at[1, pl.ds(base, DEG_CPT)], idxd_v)
    _fill_lane(bufa_v, 0)
    _fill_lane(bufb_v, 1)
    plsc.subcore_barrier()

    def body(j, carry):
        pltpu.sync_copy(bufa_v, shared.at[idxs_v.at[j]], add=True)
        pltpu.sync_copy(bufb_v, shared.at[idxd_v.at[j]], add=True)
        return carry

    lax.fori_loop(0, DEG_CPT, body, 0)
    plsc.subcore_barrier()

    for t in range(ROW_CHUNKS):
        r0 = s * ROWS_PER_TILE + t * 128
        pltpu.sync_copy(shared.at[pl.ds(r0, 128)], bufa_v)
        pltpu.sync_copy(bufa_v, deg3.at[c, pl.ds(r0, 128)])


NBUF = 2
HALF = CHUNKS_PER_TILE // 2
HGROUPS = HALF // NBUF


@functools.partial(
    pl.kernel,
    out_type=jax.ShapeDtypeStruct((NC, NPAD, DH), jnp.float32),
    mesh=_MESH,
    scratch_types=[
        pltpu.VMEM((HALF, EC), jnp.int32),
        pltpu.VMEM((HALF, EC), jnp.int32),
        pltpu.VMEM((NBUF * EC, DH), jnp.float32),
        pltpu.VMEM_SHARED((NPAD, DH), jnp.float32),
        pltpu.SemaphoreType.DMA,
        pltpu.SemaphoreType.DMA,
    ],
)
def _agg_kernel(hs3, edges3, agg3, idxs_v, idxd_v, rows_v, shared, sem0, sem1):
    c = lax.axis_index("c")
    s = lax.axis_index("s")
    sems = (sem0, sem1)

    def buf(b):
        return rows_v.at[pl.ds(b * EC, EC)]

    _fill2d(rows_v, EC, DH // LANES, 0.0)
    for t in range(ROW_CHUNKS):
        pltpu.sync_copy(buf(0), shared.at[pl.ds(s * ROWS_PER_TILE + t * 128, 128)])
    plsc.subcore_barrier()

    for h in range(2):
        base = s * CHUNKS_PER_TILE + h * HALF
        pltpu.sync_copy(edges3.at[0, pl.ds(base, HALF)], idxs_v)
        pltpu.sync_copy(edges3.at[1, pl.ds(base, HALF)], idxd_v)

        for b in range(NBUF):
            pltpu.async_copy(hs3.at[c].at[idxs_v.at[b]], buf(b), sems[b])

        def body(g, carry):
            j0 = g * NBUF
            for b in range(NBUF):
                j = j0 + b
                pltpu.make_async_copy(hs3.at[c].at[idxs_v.at[j]], buf(b), sems[b]).wait()
                pltpu.sync_copy(buf(b), shared.at[idxd_v.at[j]], add=True)
                pltpu.async_copy(hs3.at[c].at[idxs_v.at[j + NBUF]], buf(b), sems[b])
            return carry

        lax.fori_loop(0, HGROUPS - 1, body, 0)
        j0 = (HGROUPS - 1) * NBUF
        for b in range(NBUF):
            pltpu.make_async_copy(hs3.at[c].at[idxs_v.at[j0 + b]], buf(b), sems[b]).wait()
            pltpu.sync_copy(buf(b), shared.at[idxd_v.at[j0 + b]], add=True)

    plsc.subcore_barrier()

    for t in range(ROW_CHUNKS):
        r0 = s * ROWS_PER_TILE + t * 128
        pltpu.sync_copy(shared.at[pl.ds(r0, 128)], buf(0))
        pltpu.sync_copy(buf(0), agg3.at[c, pl.ds(r0, 128)])


def _pre_body(deg_ref, x_ref, ns_ref, nd_ref, hs_ref):
    do = deg_ref[0, :, 0:1] + deg_ref[1, :, 0:1]
    di = deg_ref[0, :, 1:2] + deg_ref[1, :, 1:2]
    ns = jnp.where(do > 0, lax.rsqrt(do), 0.0)
    nd = jnp.where(di > 0, lax.rsqrt(di), 0.0)
    ns_ref[...] = jnp.broadcast_to(ns, (128, DH))
    nd_ref[...] = jnp.broadcast_to(nd, (128, DH))
    hs = x_ref[...] * ns
    hs_ref[0] = hs[:, :DH]
    hs_ref[1] = hs[:, DH:]


_pre_call = pl.pallas_call(
    _pre_body,
    grid=(NPAD // 128,),
    in_specs=[
        pl.BlockSpec((NC, 128, DH), lambda i: (0, i, 0)),
        pl.BlockSpec((128, D), lambda i: (i, 0)),
    ],
    out_specs=[
        pl.BlockSpec((128, DH), lambda i: (i, 0)),
        pl.BlockSpec((128, DH), lambda i: (i, 0)),
        pl.BlockSpec((NC, 128, DH), lambda i: (0, i, 0)),
    ],
    out_shape=[
        jax.ShapeDtypeStruct((NPAD, DH), jnp.float32),
        jax.ShapeDtypeStruct((NPAD, DH), jnp.float32),
        jax.ShapeDtypeStruct((NC, NPAD, DH), jnp.float32),
    ],
)


def _layer_body(agg_ref, nd_ref, ns_ref, w_ref, b_ref, hs_ref):
    w = w_ref[...]
    t = jnp.dot(agg_ref[0], w[:DH, :], preferred_element_type=jnp.float32)
    t += jnp.dot(agg_ref[1], w[DH:, :], preferred_element_type=jnp.float32)
    t = t * nd_ref[:, :1]
    h = jnp.maximum(t + b_ref[...], 0.0)
    hs = h * ns_ref[:, :1]
    hs_ref[0] = hs[:, :DH]
    hs_ref[1] = hs[:, DH:]


_layer_call = pl.pallas_call(
    _layer_body,
    grid=(NPAD // 128,),
    in_specs=[
        pl.BlockSpec((NC, 128, DH), lambda i: (0, i, 0)),
        pl.BlockSpec((128, DH), lambda i: (i, 0)),
        pl.BlockSpec((128, DH), lambda i: (i, 0)),
        pl.BlockSpec((D, D), lambda i: (0, 0)),
        pl.BlockSpec((1, D), lambda i: (0, 0)),
    ],
    out_specs=[
        pl.BlockSpec((NC, 128, DH), lambda i: (0, i, 0)),
    ],
    out_shape=[
        jax.ShapeDtypeStruct((NC, NPAD, DH), jnp.float32),
    ],
)


def _final_body(agg_ref, nd_ref, w_ref, b_ref, h_ref, hc_ref):
    w = w_ref[...]
    t = jnp.dot(agg_ref[0], w[:DH, :], preferred_element_type=jnp.float32)
    t += jnp.dot(agg_ref[1], w[DH:, :], preferred_element_type=jnp.float32)
    t = t * nd_ref[:, :1]
    h = jnp.maximum(t + b_ref[...], 0.0)
    h_ref[...] = h
    hc_ref[...] = jnp.where(h >= 0.5, jnp.float32(1.0), jnp.float32(0.0))


_final_call = pl.pallas_call(
    _final_body,
    grid=(NPAD // 128,),
    in_specs=[
        pl.BlockSpec((NC, 128, DH), lambda i: (0, i, 0)),
        pl.BlockSpec((128, DH), lambda i: (i, 0)),
        pl.BlockSpec((D, D), lambda i: (0, 0)),
        pl.BlockSpec((1, D), lambda i: (0, 0)),
    ],
    out_specs=[
        pl.BlockSpec((128, D), lambda i: (i, 0)),
        pl.BlockSpec((128, D), lambda i: (i, 0)),
    ],
    out_shape=[
        jax.ShapeDtypeStruct((NPAD, D), jnp.float32),
        jax.ShapeDtypeStruct((NPAD, D), jnp.float32),
    ],
)


def kernel(x, edge_index, W1, W2, W3, W4, W5, b1, b2, b3, b4, b5):
    epad = jnp.full((2, EPAD - E), PAD_NODE, dtype=jnp.int32)
    edges3 = jnp.concatenate([edge_index, epad], axis=1).reshape(
        2, NS * CHUNKS_PER_TILE, EC)
    xp = jnp.pad(x, ((0, NPAD - N), (0, 0)))

    deg3 = _deg_kernel(edges3)
    ns, nd, hs3 = _pre_call(deg3, xp)

    for W, b in ((W1, b1), (W2, b2), (W3, b3), (W4, b4)):
        agg3 = _agg_kernel(hs3, edges3)
        (hs3,) = _layer_call(agg3, nd, ns, W, b.reshape(1, D))

    agg3 = _agg_kernel(hs3, edges3)
    h, hc = _final_call(agg3, nd, W5, b5.reshape(1, D))
    return h[:N], hc[:N]


# speedup vs baseline: 1.1337x; 1.0491x over previous
import functools

import jax
import jax.numpy as jnp
from jax import lax
from jax.experimental import pallas as pl
from jax.experimental.pallas import tpu as pltpu
from jax.experimental.pallas import tpu_sc as plsc

N = 10000
E = 160000
D = 256
DH = 128

NC = 2
NS = 16
LANES = 16

NPAD = 10240
ROWS_PER_TILE = NPAD // NS
ROW_CHUNKS = ROWS_PER_TILE // 128
EC = 128
CHUNKS_PER_TILE = 80
EPT = CHUNKS_PER_TILE * EC
EPAD = NS * EPT
PAD_NODE = N

_MESH = plsc.VectorSubcoreMesh(core_axis_name="c", subcore_axis_name="s")


def _fill2d(ref, nrows, ncolchunks, val):
    v = jnp.full((LANES,), val, dtype=jnp.float32)

    def body(i, carry):
        for cc in range(ncolchunks):
            ref[i, pl.ds(cc * LANES, LANES)] = v
        return carry

    lax.fori_loop(0, nrows, body, 0)


DEG_CPT = NS * CHUNKS_PER_TILE // (NC * NS)


def _fill_lane(ref, lane):
    i16 = lax.iota(jnp.int32, LANES)

    def body(i, carry):
        for cc in range(DH // LANES):
            v = jnp.where(i16 + cc * LANES == lane, jnp.float32(1.0),
                          jnp.float32(0.0))
            ref[i, pl.ds(cc * LANES, LANES)] = v
        return carry

    lax.fori_loop(0, EC, body, 0)


@functools.partial(
    pl.kernel,
    out_type=jax.ShapeDtypeStruct((NC, NPAD, DH), jnp.float32),
    mesh=_MESH,
    scratch_types=[
        pltpu.VMEM((DEG_CPT, EC), jnp.int32),
        pltpu.VMEM((DEG_CPT, EC), jnp.int32),
        pltpu.VMEM((EC, DH), jnp.float32),
        pltpu.VMEM((EC, DH), jnp.float32),
        pltpu.VMEM_SHARED((NPAD, DH), jnp.float32),
    ],
)
def _deg_kernel(edges3, deg3, idxs_v, idxd_v, bufa_v, bufb_v, shared):
    c = lax.axis_index("c")
    s = lax.axis_index("s")

    _fill2d(bufa_v, EC, DH // LANES, 0.0)
    for t in range(ROW_CHUNKS):
        pltpu.sync_copy(bufa_v, shared.at[pl.ds(s * ROWS_PER_TILE + t * 128, 128)])

    base = (c * NS + s) * DEG_CPT
    pltpu.sync_copy(edges3.at[0, pl.ds(base, DEG_CPT)], idxs_v)
    pltpu.sync_copy(edges3.at[1, pl.ds(base, DEG_CPT)], idxd_v)
    _fill_lane(bufa_v, 0)
    _fill_lane(bufb_v, 1)
    plsc.subcore_barrier()

    def body(j, carry):
        pltpu.sync_copy(bufa_v, shared.at[idxs_v.at[j]], add=True)
        pltpu.sync_copy(bufb_v, shared.at[idxd_v.at[j]], add=True)
        return carry

    lax.fori_loop(0, DEG_CPT, body, 0)
    plsc.subcore_barrier()

    for t in range(ROW_CHUNKS):
        r0 = s * ROWS_PER_TILE + t * 128
        pltpu.sync_copy(shared.at[pl.ds(r0, 128)], bufa_v)
        pltpu.sync_copy(bufa_v, deg3.at[c, pl.ds(r0, 128)])


NBUF = 2
HALF = CHUNKS_PER_TILE // 2
HGROUPS = HALF // NBUF


@functools.partial(
    pl.kernel,
    out_type=jax.ShapeDtypeStruct((NC, NPAD, DH), jnp.float32),
    mesh=_MESH,
    scratch_types=[
        pltpu.VMEM((HALF, EC), jnp.int32),
        pltpu.VMEM((HALF, EC), jnp.int32),
        pltpu.VMEM((NBUF * EC, DH), jnp.float32),
        pltpu.VMEM_SHARED((NPAD, DH), jnp.float32),
        pltpu.SemaphoreType.DMA,
        pltpu.SemaphoreType.DMA,
    ],
)
def _agg_kernel(hs3, edges3, agg3, idxs_v, idxd_v, rows_v, shared, sem0, sem1):
    c = lax.axis_index("c")
    s = lax.axis_index("s")
    sems = (sem0, sem1)

    def buf(b):
        return rows_v.at[pl.ds(b * EC, EC)]

    _fill2d(rows_v, EC, DH // LANES, 0.0)
    for t in range(ROW_CHUNKS):
        pltpu.sync_copy(buf(0), shared.at[pl.ds(s * ROWS_PER_TILE + t * 128, 128)])
    plsc.subcore_barrier()

    for h in range(2):
        base = s * CHUNKS_PER_TILE + h * HALF
        pltpu.sync_copy(edges3.at[0, pl.ds(base, HALF)], idxs_v)
        pltpu.sync_copy(edges3.at[1, pl.ds(base, HALF)], idxd_v)

        for b in range(NBUF):
            pltpu.async_copy(hs3.at[c].at[idxs_v.at[b]], buf(b), sems[b])

        def body(g, carry):
            j0 = g * NBUF
            for b in range(NBUF):
                j = j0 + b
                pltpu.make_async_copy(hs3.at[c].at[idxs_v.at[j]], buf(b), sems[b]).wait()
                pltpu.sync_copy(buf(b), shared.at[idxd_v.at[j]], add=True)
                pltpu.async_copy(hs3.at[c].at[idxs_v.at[j + NBUF]], buf(b), sems[b])
            return carry

        lax.fori_loop(0, HGROUPS - 1, body, 0)
        j0 = (HGROUPS - 1) * NBUF
        for b in range(NBUF):
            pltpu.make_async_copy(hs3.at[c].at[idxs_v.at[j0 + b]], buf(b), sems[b]).wait()
            pltpu.sync_copy(buf(b), shared.at[idxd_v.at[j0 + b]], add=True)

    plsc.subcore_barrier()

    for t in range(ROW_CHUNKS):
        r0 = s * ROWS_PER_TILE + t * 128
        pltpu.sync_copy(shared.at[pl.ds(r0, 128)], buf(0))
        pltpu.sync_copy(buf(0), agg3.at[c, pl.ds(r0, 128)])


BR = 512


def _pre_body(deg_ref, x_ref, nrm_ref, hs_ref):
    do = deg_ref[0, :, 0:1] + deg_ref[1, :, 0:1]
    di = deg_ref[0, :, 1:2] + deg_ref[1, :, 1:2]
    ns = jnp.where(do > 0, lax.rsqrt(do), 0.0)
    nd = jnp.where(di > 0, lax.rsqrt(di), 0.0)
    lane = lax.broadcasted_iota(jnp.int32, (BR, DH), 1)
    nrm_ref[...] = jnp.where(lane == 0, ns, jnp.where(lane == 1, nd, 0.0))
    hs = x_ref[...] * ns
    hs_ref[0] = hs[:, :DH]
    hs_ref[1] = hs[:, DH:]


_pre_call = pl.pallas_call(
    _pre_body,
    grid=(NPAD // BR,),
    in_specs=[
        pl.BlockSpec((NC, BR, DH), lambda i: (0, i, 0)),
        pl.BlockSpec((BR, D), lambda i: (i, 0)),
    ],
    out_specs=[
        pl.BlockSpec((BR, DH), lambda i: (i, 0)),
        pl.BlockSpec((NC, BR, DH), lambda i: (0, i, 0)),
    ],
    out_shape=[
        jax.ShapeDtypeStruct((NPAD, DH), jnp.float32),
        jax.ShapeDtypeStruct((NC, NPAD, DH), jnp.float32),
    ],
)


def _layer_body(agg_ref, nrm_ref, w_ref, b_ref, hs_ref):
    w = w_ref[...]
    t = jnp.dot(agg_ref[0], w[:DH, :], preferred_element_type=jnp.float32)
    t += jnp.dot(agg_ref[1], w[DH:, :], preferred_element_type=jnp.float32)
    t = t * nrm_ref[:, 1:2]
    h = jnp.maximum(t + b_ref[...], 0.0)
    hs = h * nrm_ref[:, 0:1]
    hs_ref[0] = hs[:, :DH]
    hs_ref[1] = hs[:, DH:]


_layer_call = pl.pallas_call(
    _layer_body,
    grid=(NPAD // BR,),
    in_specs=[
        pl.BlockSpec((NC, BR, DH), lambda i: (0, i, 0)),
        pl.BlockSpec((BR, DH), lambda i: (i, 0)),
        pl.BlockSpec((D, D), lambda i: (0, 0)),
        pl.BlockSpec((1, D), lambda i: (0, 0)),
    ],
    out_specs=[
        pl.BlockSpec((NC, BR, DH), lambda i: (0, i, 0)),
    ],
    out_shape=[
        jax.ShapeDtypeStruct((NC, NPAD, DH), jnp.float32),
    ],
)


def _final_body(agg_ref, nrm_ref, w_ref, b_ref, h_ref, hc_ref):
    w = w_ref[...]
    t = jnp.dot(agg_ref[0], w[:DH, :], preferred_element_type=jnp.float32)
    t += jnp.dot(agg_ref[1], w[DH:, :], preferred_element_type=jnp.float32)
    t = t * nrm_ref[:, 1:2]
    h = jnp.maximum(t + b_ref[...], 0.0)
    h_ref[...] = h
    hc_ref[...] = jnp.where(h >= 0.5, jnp.float32(1.0), jnp.float32(0.0))


_final_call = pl.pallas_call(
    _final_body,
    grid=(NPAD // BR,),
    in_specs=[
        pl.BlockSpec((NC, BR, DH), lambda i: (0, i, 0)),
        pl.BlockSpec((BR, DH), lambda i: (i, 0)),
        pl.BlockSpec((D, D), lambda i: (0, 0)),
        pl.BlockSpec((1, D), lambda i: (0, 0)),
    ],
    out_specs=[
        pl.BlockSpec((BR, D), lambda i: (i, 0)),
        pl.BlockSpec((BR, D), lambda i: (i, 0)),
    ],
    out_shape=[
        jax.ShapeDtypeStruct((NPAD, D), jnp.float32),
        jax.ShapeDtypeStruct((NPAD, D), jnp.float32),
    ],
)


def kernel(x, edge_index, W1, W2, W3, W4, W5, b1, b2, b3, b4, b5):
    epad = jnp.full((2, EPAD - E), PAD_NODE, dtype=jnp.int32)
    edges3 = jnp.concatenate([edge_index, epad], axis=1).reshape(
        2, NS * CHUNKS_PER_TILE, EC)
    xp = jnp.pad(x, ((0, NPAD - N), (0, 0)))

    deg3 = _deg_kernel(edges3)
    nrm, hs3 = _pre_call(deg3, xp)

    for W, b in ((W1, b1), (W2, b2), (W3, b3), (W4, b4)):
        agg3 = _agg_kernel(hs3, edges3)
        (hs3,) = _layer_call(agg3, nrm, W, b.reshape(1, D))

    agg3 = _agg_kernel(hs3, edges3)
    h, hc = _final_call(agg3, nrm, W5, b5.reshape(1, D))
    return h[:N], hc[:N]


# speedup vs baseline: 1.1580x; 1.0214x over previous
import functools

import jax
import jax.numpy as jnp
from jax import lax
from jax.experimental import pallas as pl
from jax.experimental.pallas import tpu as pltpu
from jax.experimental.pallas import tpu_sc as plsc

N = 10000
E = 160000
D = 256
DH = 128

NC = 2
NS = 16
LANES = 16

NPAD = 10240
ROWS_PER_TILE = NPAD // NS
ROW_CHUNKS = ROWS_PER_TILE // 128
EC = 128
CHUNKS_PER_TILE = 80
EPT = CHUNKS_PER_TILE * EC
EPAD = NS * EPT
PAD_NODE = N

_MESH = plsc.VectorSubcoreMesh(core_axis_name="c", subcore_axis_name="s")


def _fill2d(ref, nrows, ncolchunks, val):
    v = jnp.full((LANES,), val, dtype=jnp.float32)

    def body(i, carry):
        for cc in range(ncolchunks):
            ref[i, pl.ds(cc * LANES, LANES)] = v
        return carry

    lax.fori_loop(0, nrows, body, 0)


DEG_CPT = NS * CHUNKS_PER_TILE // (NC * NS)


def _fill_lane(ref, lane):
    i16 = lax.iota(jnp.int32, LANES)

    def body(i, carry):
        for cc in range(DH // LANES):
            v = jnp.where(i16 + cc * LANES == lane, jnp.float32(1.0),
                          jnp.float32(0.0))
            ref[i, pl.ds(cc * LANES, LANES)] = v
        return carry

    lax.fori_loop(0, EC, body, 0)


@functools.partial(
    pl.kernel,
    out_type=jax.ShapeDtypeStruct((NC, NPAD, DH), jnp.float32),
    mesh=_MESH,
    scratch_types=[
        pltpu.VMEM((DEG_CPT, EC), jnp.int32),
        pltpu.VMEM((DEG_CPT, EC), jnp.int32),
        pltpu.VMEM((EC, DH), jnp.float32),
        pltpu.VMEM((EC, DH), jnp.float32),
        pltpu.VMEM_SHARED((NPAD, DH), jnp.float32),
    ],
)
def _deg_kernel(edges3, deg3, idxs_v, idxd_v, bufa_v, bufb_v, shared):
    c = lax.axis_index("c")
    s = lax.axis_index("s")

    _fill2d(bufa_v, EC, DH // LANES, 0.0)
    for t in range(ROW_CHUNKS):
        pltpu.sync_copy(bufa_v, shared.at[pl.ds(s * ROWS_PER_TILE + t * 128, 128)])

    base = (c * NS + s) * DEG_CPT
    pltpu.sync_copy(edges3.at[0, pl.ds(base, DEG_CPT)], idxs_v)
    pltpu.sync_copy(edges3.at[1, pl.ds(base, DEG_CPT)], idxd_v)
    _fill_lane(bufa_v, 0)
    _fill_lane(bufb_v, 1)
    plsc.subcore_barrier()

    def body(j, carry):
        pltpu.sync_copy(bufa_v, shared.at[idxs_v.at[j]], add=True)
        pltpu.sync_copy(bufb_v, shared.at[idxd_v.at[j]], add=True)
        return carry

    lax.fori_loop(0, DEG_CPT, body, 0)
    plsc.subcore_barrier()

    for t in range(ROW_CHUNKS):
        r0 = s * ROWS_PER_TILE + t * 128
        pltpu.sync_copy(shared.at[pl.ds(r0, 128)], bufa_v)
        pltpu.sync_copy(bufa_v, deg3.at[c, pl.ds(r0, 128)])


NBUF = 2
HALF = CHUNKS_PER_TILE // 2
HGROUPS = HALF // NBUF


@functools.partial(
    pl.kernel,
    out_type=jax.ShapeDtypeStruct((NC, NPAD, DH), jnp.float32),
    mesh=_MESH,
    scratch_types=[
        pltpu.VMEM((HALF, EC), jnp.int32),
        pltpu.VMEM((HALF, EC), jnp.int32),
        pltpu.VMEM((NBUF * EC, DH), jnp.float32),
        pltpu.VMEM_SHARED((NPAD, DH), jnp.float32),
        pltpu.SemaphoreType.DMA,
        pltpu.SemaphoreType.DMA,
    ],
)
def _agg_kernel(hs3, edges3, agg3, idxs_v, idxd_v, rows_v, shared, sem0, sem1):
    c = lax.axis_index("c")
    s = lax.axis_index("s")
    sems = (sem0, sem1)

    def buf(b):
        return rows_v.at[pl.ds(b * EC, EC)]

    _fill2d(rows_v, EC, DH // LANES, 0.0)
    for t in range(ROW_CHUNKS):
        pltpu.sync_copy(buf(0), shared.at[pl.ds(s * ROWS_PER_TILE + t * 128, 128)])
    plsc.subcore_barrier()

    for h in range(2):
        base = s * CHUNKS_PER_TILE + h * HALF
        pltpu.sync_copy(edges3.at[0, pl.ds(base, HALF)], idxs_v)
        pltpu.sync_copy(edges3.at[1, pl.ds(base, HALF)], idxd_v)

        for b in range(NBUF):
            pltpu.async_copy(hs3.at[c].at[idxs_v.at[b]], buf(b), sems[b])

        def body(g, carry):
            j0 = g * NBUF
            for b in range(NBUF):
                j = j0 + b
                pltpu.make_async_copy(hs3.at[c].at[idxs_v.at[j]], buf(b), sems[b]).wait()
                pltpu.sync_copy(buf(b), shared.at[idxd_v.at[j]], add=True)
                pltpu.async_copy(hs3.at[c].at[idxs_v.at[j + NBUF]], buf(b), sems[b])
            return carry

        lax.fori_loop(0, HGROUPS - 1, body, 0)
        j0 = (HGROUPS - 1) * NBUF
        for b in range(NBUF):
            pltpu.make_async_copy(hs3.at[c].at[idxs_v.at[j0 + b]], buf(b), sems[b]).wait()
            pltpu.sync_copy(buf(b), shared.at[idxd_v.at[j0 + b]], add=True)

    plsc.subcore_barrier()

    for t in range(ROW_CHUNKS):
        r0 = s * ROWS_PER_TILE + t * 128
        pltpu.sync_copy(shared.at[pl.ds(r0, 128)], buf(0))
        pltpu.sync_copy(buf(0), agg3.at[c, pl.ds(r0, 128)])


BR = 1024


def _pre_body(deg_ref, x_ref, nrm_ref, hs_ref):
    do = deg_ref[0, :, 0:1] + deg_ref[1, :, 0:1]
    di = deg_ref[0, :, 1:2] + deg_ref[1, :, 1:2]
    ns = jnp.where(do > 0, lax.rsqrt(do), 0.0)
    nd = jnp.where(di > 0, lax.rsqrt(di), 0.0)
    lane = lax.broadcasted_iota(jnp.int32, (BR, DH), 1)
    nrm_ref[...] = jnp.where(lane == 0, ns, jnp.where(lane == 1, nd, 0.0))
    hs = x_ref[...] * ns
    hs_ref[0] = hs[:, :DH]
    hs_ref[1] = hs[:, DH:]


_pre_call = pl.pallas_call(
    _pre_body,
    grid=(NPAD // BR,),
    in_specs=[
        pl.BlockSpec((NC, BR, DH), lambda i: (0, i, 0)),
        pl.BlockSpec((BR, D), lambda i: (i, 0)),
    ],
    out_specs=[
        pl.BlockSpec((BR, DH), lambda i: (i, 0)),
        pl.BlockSpec((NC, BR, DH), lambda i: (0, i, 0)),
    ],
    out_shape=[
        jax.ShapeDtypeStruct((NPAD, DH), jnp.float32),
        jax.ShapeDtypeStruct((NC, NPAD, DH), jnp.float32),
    ],
)


def _layer_body(agg_ref, nrm_ref, w_ref, b_ref, hs_ref):
    w = w_ref[...]
    t = jnp.dot(agg_ref[0], w[:DH, :], preferred_element_type=jnp.float32)
    t += jnp.dot(agg_ref[1], w[DH:, :], preferred_element_type=jnp.float32)
    t = t * nrm_ref[:, 1:2]
    h = jnp.maximum(t + b_ref[...], 0.0)
    hs = h * nrm_ref[:, 0:1]
    hs_ref[0] = hs[:, :DH]
    hs_ref[1] = hs[:, DH:]


_layer_call = pl.pallas_call(
    _layer_body,
    grid=(NPAD // BR,),
    in_specs=[
        pl.BlockSpec((NC, BR, DH), lambda i: (0, i, 0)),
        pl.BlockSpec((BR, DH), lambda i: (i, 0)),
        pl.BlockSpec((D, D), lambda i: (0, 0)),
        pl.BlockSpec((1, D), lambda i: (0, 0)),
    ],
    out_specs=[
        pl.BlockSpec((NC, BR, DH), lambda i: (0, i, 0)),
    ],
    out_shape=[
        jax.ShapeDtypeStruct((NC, NPAD, DH), jnp.float32),
    ],
)


def _final_body(agg_ref, nrm_ref, w_ref, b_ref, h_ref, hc_ref):
    w = w_ref[...]
    t = jnp.dot(agg_ref[0], w[:DH, :], preferred_element_type=jnp.float32)
    t += jnp.dot(agg_ref[1], w[DH:, :], preferred_element_type=jnp.float32)
    t = t * nrm_ref[:, 1:2]
    h = jnp.maximum(t + b_ref[...], 0.0)
    h_ref[...] = h
    hc_ref[...] = jnp.where(h >= 0.5, jnp.float32(1.0), jnp.float32(0.0))


_final_call = pl.pallas_call(
    _final_body,
    grid=(NPAD // BR,),
    in_specs=[
        pl.BlockSpec((NC, BR, DH), lambda i: (0, i, 0)),
        pl.BlockSpec((BR, DH), lambda i: (i, 0)),
        pl.BlockSpec((D, D), lambda i: (0, 0)),
        pl.BlockSpec((1, D), lambda i: (0, 0)),
    ],
    out_specs=[
        pl.BlockSpec((BR, D), lambda i: (i, 0)),
        pl.BlockSpec((BR, D), lambda i: (i, 0)),
    ],
    out_shape=[
        jax.ShapeDtypeStruct((NPAD, D), jnp.float32),
        jax.ShapeDtypeStruct((NPAD, D), jnp.float32),
    ],
)


def kernel(x, edge_index, W1, W2, W3, W4, W5, b1, b2, b3, b4, b5):
    epad = jnp.full((2, EPAD - E), PAD_NODE, dtype=jnp.int32)
    edges3 = jnp.concatenate([edge_index, epad], axis=1).reshape(
        2, NS * CHUNKS_PER_TILE, EC)
    xp = jnp.pad(x, ((0, NPAD - N), (0, 0)))

    deg3 = _deg_kernel(edges3)
    nrm, hs3 = _pre_call(deg3, xp)

    for W, b in ((W1, b1), (W2, b2), (W3, b3), (W4, b4)):
        agg3 = _agg_kernel(hs3, edges3)
        (hs3,) = _layer_call(agg3, nrm, W, b.reshape(1, D))

    agg3 = _agg_kernel(hs3, edges3)
    h, hc = _final_call(agg3, nrm, W5, b5.reshape(1, D))
    return h[:N], hc[:N]


# speedup vs baseline: 1.1644x; 1.0055x over previous
import functools

import jax
import jax.numpy as jnp
from jax import lax
from jax.experimental import pallas as pl
from jax.experimental.pallas import tpu as pltpu
from jax.experimental.pallas import tpu_sc as plsc

N = 10000
E = 160000
D = 256
DH = 128

NC = 2
NS = 16
LANES = 16

NPAD = 10240
ROWS_PER_TILE = NPAD // NS
ROW_CHUNKS = ROWS_PER_TILE // 128
EC = 128
CHUNKS_PER_TILE = 80
EPT = CHUNKS_PER_TILE * EC
EPAD = NS * EPT
PAD_NODE = N

_MESH = plsc.VectorSubcoreMesh(core_axis_name="c", subcore_axis_name="s")


def _fill2d(ref, nrows, ncolchunks, val):
    v = jnp.full((LANES,), val, dtype=jnp.float32)

    def body(i, carry):
        for cc in range(ncolchunks):
            ref[i, pl.ds(cc * LANES, LANES)] = v
        return carry

    lax.fori_loop(0, nrows, body, 0)


DEG_CPT = NS * CHUNKS_PER_TILE // (NC * NS)


def _fill_lane(ref, lane):
    i16 = lax.iota(jnp.int32, LANES)

    def body(i, carry):
        for cc in range(DH // LANES):
            v = jnp.where(i16 + cc * LANES == lane, jnp.float32(1.0),
                          jnp.float32(0.0))
            ref[i, pl.ds(cc * LANES, LANES)] = v
        return carry

    lax.fori_loop(0, EC, body, 0)


@functools.partial(
    pl.kernel,
    out_type=jax.ShapeDtypeStruct((NC, NPAD, DH), jnp.float32),
    mesh=_MESH,
    scratch_types=[
        pltpu.VMEM((DEG_CPT, EC), jnp.int32),
        pltpu.VMEM((DEG_CPT, EC), jnp.int32),
        pltpu.VMEM((EC, DH), jnp.float32),
        pltpu.VMEM((EC, DH), jnp.float32),
        pltpu.VMEM_SHARED((NPAD, DH), jnp.float32),
    ],
)
def _deg_kernel(edges3, deg3, idxs_v, idxd_v, bufa_v, bufb_v, shared):
    c = lax.axis_index("c")
    s = lax.axis_index("s")

    _fill2d(bufa_v, EC, DH // LANES, 0.0)
    for t in range(ROW_CHUNKS):
        pltpu.sync_copy(bufa_v, shared.at[pl.ds(s * ROWS_PER_TILE + t * 128, 128)])

    base = (c * NS + s) * DEG_CPT
    pltpu.sync_copy(edges3.at[0, pl.ds(base, DEG_CPT)], idxs_v)
    pltpu.sync_copy(edges3.at[1, pl.ds(base, DEG_CPT)], idxd_v)
    _fill_lane(bufa_v, 0)
    _fill_lane(bufb_v, 1)
    plsc.subcore_barrier()

    def body(j, carry):
        pltpu.sync_copy(bufa_v, shared.at[idxs_v.at[j]], add=True)
        pltpu.sync_copy(bufb_v, shared.at[idxd_v.at[j]], add=True)
        return carry

    lax.fori_loop(0, DEG_CPT, body, 0)
    plsc.subcore_barrier()

    for t in range(ROW_CHUNKS):
        r0 = s * ROWS_PER_TILE + t * 128
        pltpu.sync_copy(shared.at[pl.ds(r0, 128)], bufa_v)
        pltpu.sync_copy(bufa_v, deg3.at[c, pl.ds(r0, 128)])


NBUF = 2
HALF = CHUNKS_PER_TILE // 2
HGROUPS = HALF // NBUF


@functools.partial(
    pl.kernel,
    out_type=jax.ShapeDtypeStruct((NC, NPAD, DH), jnp.float32),
    mesh=_MESH,
    scratch_types=[
        pltpu.VMEM((HALF, EC), jnp.int32),
        pltpu.VMEM((HALF, EC), jnp.int32),
        pltpu.VMEM((NBUF * EC, DH), jnp.float32),
        pltpu.VMEM_SHARED((NPAD, DH), jnp.float32),
        pltpu.SemaphoreType.DMA,
        pltpu.SemaphoreType.DMA,
    ],
)
def _agg_kernel(hs3, edges3, agg3, idxs_v, idxd_v, rows_v, shared, sem0, sem1):
    c = lax.axis_index("c")
    s = lax.axis_index("s")
    sems = (sem0, sem1)

    def buf(b):
        return rows_v.at[pl.ds(b * EC, EC)]

    _fill2d(rows_v, EC, DH // LANES, 0.0)
    for t in range(ROW_CHUNKS):
        pltpu.sync_copy(buf(0), shared.at[pl.ds(s * ROWS_PER_TILE + t * 128, 128)])
    plsc.subcore_barrier()

    for h in range(2):
        base = s * CHUNKS_PER_TILE + h * HALF
        pltpu.sync_copy(edges3.at[0, pl.ds(base, HALF)], idxs_v)
        pltpu.sync_copy(edges3.at[1, pl.ds(base, HALF)], idxd_v)

        for b in range(NBUF):
            pltpu.async_copy(hs3.at[c].at[idxs_v.at[b]], buf(b), sems[b])

        def body(g, carry):
            j0 = g * NBUF
            for b in range(NBUF):
                j = j0 + b
                pltpu.make_async_copy(hs3.at[c].at[idxs_v.at[j]], buf(b), sems[b]).wait()
                pltpu.sync_copy(buf(b), shared.at[idxd_v.at[j]], add=True)
                pltpu.async_copy(hs3.at[c].at[idxs_v.at[j + NBUF]], buf(b), sems[b])
            return carry

        lax.fori_loop(0, HGROUPS - 1, body, 0)
        j0 = (HGROUPS - 1) * NBUF
        for b in range(NBUF):
            pltpu.make_async_copy(hs3.at[c].at[idxs_v.at[j0 + b]], buf(b), sems[b]).wait()
            pltpu.sync_copy(buf(b), shared.at[idxd_v.at[j0 + b]], add=True)

    plsc.subcore_barrier()

    for t in range(ROW_CHUNKS):
        r0 = s * ROWS_PER_TILE + t * 128
        pltpu.sync_copy(shared.at[pl.ds(r0, 128)], buf(0))
        pltpu.sync_copy(buf(0), agg3.at[c, pl.ds(r0, 128)])


BR = 2048


def _pre_body(deg_ref, x_ref, nrm_ref, hs_ref):
    do = deg_ref[0, :, 0:1] + deg_ref[1, :, 0:1]
    di = deg_ref[0, :, 1:2] + deg_ref[1, :, 1:2]
    ns = jnp.where(do > 0, lax.rsqrt(do), 0.0)
    nd = jnp.where(di > 0, lax.rsqrt(di), 0.0)
    lane = lax.broadcasted_iota(jnp.int32, (BR, DH), 1)
    nrm_ref[...] = jnp.where(lane == 0, ns, jnp.where(lane == 1, nd, 0.0))
    hs = x_ref[...] * ns
    hs_ref[0] = hs[:, :DH]
    hs_ref[1] = hs[:, DH:]


_pre_call = pl.pallas_call(
    _pre_body,
    grid=(NPAD // BR,),
    in_specs=[
        pl.BlockSpec((NC, BR, DH), lambda i: (0, i, 0)),
        pl.BlockSpec((BR, D), lambda i: (i, 0)),
    ],
    out_specs=[
        pl.BlockSpec((BR, DH), lambda i: (i, 0)),
        pl.BlockSpec((NC, BR, DH), lambda i: (0, i, 0)),
    ],
    out_shape=[
        jax.ShapeDtypeStruct((NPAD, DH), jnp.float32),
        jax.ShapeDtypeStruct((NC, NPAD, DH), jnp.float32),
    ],
)


def _layer_body(agg_ref, nrm_ref, w_ref, b_ref, hs_ref):
    w = w_ref[...]
    t = jnp.dot(agg_ref[0], w[:DH, :], preferred_element_type=jnp.float32)
    t += jnp.dot(agg_ref[1], w[DH:, :], preferred_element_type=jnp.float32)
    t = t * nrm_ref[:, 1:2]
    h = jnp.maximum(t + b_ref[...], 0.0)
    hs = h * nrm_ref[:, 0:1]
    hs_ref[0] = hs[:, :DH]
    hs_ref[1] = hs[:, DH:]


_layer_call = pl.pallas_call(
    _layer_body,
    grid=(NPAD // BR,),
    in_specs=[
        pl.BlockSpec((NC, BR, DH), lambda i: (0, i, 0)),
        pl.BlockSpec((BR, DH), lambda i: (i, 0)),
        pl.BlockSpec((D, D), lambda i: (0, 0)),
        pl.BlockSpec((1, D), lambda i: (0, 0)),
    ],
    out_specs=[
        pl.BlockSpec((NC, BR, DH), lambda i: (0, i, 0)),
    ],
    out_shape=[
        jax.ShapeDtypeStruct((NC, NPAD, DH), jnp.float32),
    ],
)


def _final_body(agg_ref, nrm_ref, w_ref, b_ref, h_ref, hc_ref):
    w = w_ref[...]
    t = jnp.dot(agg_ref[0], w[:DH, :], preferred_element_type=jnp.float32)
    t += jnp.dot(agg_ref[1], w[DH:, :], preferred_element_type=jnp.float32)
    t = t * nrm_ref[:, 1:2]
    h = jnp.maximum(t + b_ref[...], 0.0)
    h_ref[...] = h
    hc_ref[...] = jnp.where(h >= 0.5, jnp.float32(1.0), jnp.float32(0.0))


_final_call = pl.pallas_call(
    _final_body,
    grid=(NPAD // BR,),
    in_specs=[
        pl.BlockSpec((NC, BR, DH), lambda i: (0, i, 0)),
        pl.BlockSpec((BR, DH), lambda i: (i, 0)),
        pl.BlockSpec((D, D), lambda i: (0, 0)),
        pl.BlockSpec((1, D), lambda i: (0, 0)),
    ],
    out_specs=[
        pl.BlockSpec((BR, D), lambda i: (i, 0)),
        pl.BlockSpec((BR, D), lambda i: (i, 0)),
    ],
    out_shape=[
        jax.ShapeDtypeStruct((NPAD, D), jnp.float32),
        jax.ShapeDtypeStruct((NPAD, D), jnp.float32),
    ],
)


def kernel(x, edge_index, W1, W2, W3, W4, W5, b1, b2, b3, b4, b5):
    epad = jnp.full((2, EPAD - E), PAD_NODE, dtype=jnp.int32)
    edges3 = jnp.concatenate([edge_index, epad], axis=1).reshape(
        2, NS * CHUNKS_PER_TILE, EC)
    xp = jnp.pad(x, ((0, NPAD - N), (0, 0)))

    deg3 = _deg_kernel(edges3)
    nrm, hs3 = _pre_call(deg3, xp)

    for W, b in ((W1, b1), (W2, b2), (W3, b3), (W4, b4)):
        agg3 = _agg_kernel(hs3, edges3)
        (hs3,) = _layer_call(agg3, nrm, W, b.reshape(1, D))

    agg3 = _agg_kernel(hs3, edges3)
    h, hc = _final_call(agg3, nrm, W5, b5.reshape(1, D))
    return h[:N], hc[:N]
